# baseline (speedup 1.0000x reference)
import jax
import jax.numpy as jnp
from jax import lax
from jax.experimental import pallas as pl
from jax.experimental.pallas import tpu as pltpu
from jax.experimental.pallas import tpu_sc as plsc

N_NODES = 10000
N_EDGES = 320000
D_FEAT = 128
HC = 64
NPAD = 10112
NW = 32
B = 64
CH = 164
EP = NW * CH * B
ROWS_PER_TILE = NPAD // 16
RPT_FULL = ROWS_PER_TILE // B
RPT_REM = ROWS_PER_TILE % B


def _lane_perm(x, idx):
    dn = lax.GatherDimensionNumbers(
        offset_dims=(), collapsed_slice_dims=(0,), start_index_map=(0,))
    return lax.gather(x, idx.reshape(16, 1), dn, (1,),
                      mode=lax.GatherScatterMode.PROMISE_IN_BOUNDS)


def _proj1_body(x_ref, wl_ref, bl_ref, wr_ref, br_ref, xl_ref, xr_ref):
    x = x_ref[...]
    xl_ref[...] = jnp.dot(x, wl_ref[...].T,
                          preferred_element_type=jnp.float32) + bl_ref[...]
    xr_ref[...] = jnp.dot(x, wr_ref[...].T,
                          preferred_element_type=jnp.float32) + br_ref[...]


def _proj1(x_pad, wl, bl, wr, br):
    blk = 632
    grid = NPAD // blk
    return pl.pallas_call(
        _proj1_body,
        grid=(grid,),
        in_specs=[
            pl.BlockSpec((blk, D_FEAT), lambda i: (i, 0)),
            pl.BlockSpec((128, D_FEAT), lambda i: (0, 0)),
            pl.BlockSpec((1, 128), lambda i: (0, 0)),
            pl.BlockSpec((128, D_FEAT), lambda i: (0, 0)),
            pl.BlockSpec((1, 128), lambda i: (0, 0)),
        ],
        out_specs=[
            pl.BlockSpec((blk, 128), lambda i: (i, 0)),
            pl.BlockSpec((blk, 128), lambda i: (i, 0)),
        ],
        out_shape=[
            jax.ShapeDtypeStruct((NPAD, 128), jnp.float32),
            jax.ShapeDtypeStruct((NPAD, 128), jnp.float32),
        ],
    )(x_pad, wl, bl.reshape(1, 128), wr, br.reshape(1, 128))


def _sc_edge_body(compute,
                  xl_hbm, xr_hbm, att_hbm, sd_hbm, out_hbm,
                  sdv0, sdv1, srcv0, dstv0, srcv1, dstv1,
                  xlb0, xrb0, xlb1, xrb1, val0, val1,
                  attb, acc, g0, g1, s0, s1, i0, i1):
    cid = lax.axis_index("c")
    sid = lax.axis_index("s")
    wid = cid * 16 + sid

    sdv = (sdv0, sdv1)
    srcv = (srcv0, srcv1)
    dstv = (dstv0, dstv1)
    xlb = (xlb0, xlb1)
    xrb = (xrb0, xrb1)
    val = (val0, val1)
    g = (g0, g1)
    s = (s0, s1)
    ih = (i0, i1)

    zeros16 = jnp.zeros((16,), jnp.float32)

    def zrow(i, _):
        for kk in range(8):
            val0[i, pl.ds(kk * 16, 16)] = zeros16
            val1[i, pl.ds(kk * 16, 16)] = zeros16
        return 0
    lax.fori_loop(0, B, zrow, 0)
    for i in range(RPT_FULL):
        pltpu.sync_copy(val0, acc.at[pl.ds(sid * ROWS_PER_TILE + i * B, B)])
    pltpu.sync_copy(
        val0.at[pl.ds(0, RPT_REM)],
        acc.at[pl.ds(sid * ROWS_PER_TILE + RPT_FULL * B, RPT_REM)])
    pltpu.sync_copy(att_hbm, attb)
    plsc.subcore_barrier()

    ebase = wid * (CH * B)

    def unpack_and_gather(pb):
        for kk in range(B // 16):
            w = sdv[pb][pl.ds(kk * 16, 16)]
            srcv[pb][pl.ds(kk * 16, 16)] = w & 16383
            dstv[pb][pl.ds(kk * 16, 16)] = w >> 14
        pltpu.async_copy(xl_hbm.at[srcv[pb]], xlb[pb], g[pb])
        pltpu.async_copy(xr_hbm.at[dstv[pb]], xrb[pb], g[pb])

    def step(j, pb):
        q = 1 - pb
        @pl.when(j >= 1)
        def _():
            pltpu.make_async_copy(val[q], acc.at[dstv[q]], s[q]).wait()

        @pl.when(j + 1 < CH)
        def _():
            pltpu.make_async_copy(
                sd_hbm.at[pl.ds(ebase + (j + 1) * B, B)], sdv[q],
                ih[q]).wait()
            unpack_and_gather(q)

        @pl.when(j + 2 < CH)
        def _():
            pltpu.async_copy(
                sd_hbm.at[pl.ds(ebase + (j + 2) * B, B)], sdv[pb], ih[pb])

        pltpu.make_async_copy(xl_hbm.at[srcv[pb]], xlb[pb], g[pb]).wait()
        pltpu.make_async_copy(xr_hbm.at[dstv[pb]], xrb[pb], g[pb]).wait()
        lax.fori_loop(
            0, B, lambda b, _: compute(b, xlb[pb], xrb[pb], val[pb], attb), 0)
        pltpu.async_copy(val[pb], acc.at[dstv[pb]], s[pb], add=True)

    pltpu.sync_copy(sd_hbm.at[pl.ds(ebase, B)], sdv[0])
    unpack_and_gather(0)
    pltpu.async_copy(sd_hbm.at[pl.ds(ebase + B, B)], sdv[1], ih[1])

    def pair(t, _):
        step(2 * t, 0)
        step(2 * t + 1, 1)
        return 0
    lax.fori_loop(0, CH // 2, pair, 0)
    pltpu.make_async_copy(val[1], acc.at[dstv[1]], s[1]).wait()

    plsc.subcore_barrier()
    for i in range(RPT_FULL):
        pltpu.sync_copy(
            acc.at[pl.ds(sid * ROWS_PER_TILE + i * B, B)],
            out_hbm.at[cid, pl.ds(sid * ROWS_PER_TILE + i * B, B)])
    pltpu.sync_copy(
        acc.at[pl.ds(sid * ROWS_PER_TILE + RPT_FULL * B, RPT_REM)],
        out_hbm.at[cid, pl.ds(sid * ROWS_PER_TILE + RPT_FULL * B, RPT_REM)])


def _sc_scratch(att_len):
    return [
        pltpu.VMEM((B,), jnp.int32),
        pltpu.VMEM((B,), jnp.int32),
        pltpu.VMEM((B,), jnp.int32),
        pltpu.VMEM((B,), jnp.int32),
        pltpu.VMEM((B,), jnp.int32),
        pltpu.VMEM((B,), jnp.int32),
        pltpu.VMEM((B, 128), jnp.float32),
        pltpu.VMEM((B, 128), jnp.float32),
        pltpu.VMEM((B, 128), jnp.float32),
        pltpu.VMEM((B, 128), jnp.float32),
        pltpu.VMEM((B, 128), jnp.float32),
        pltpu.VMEM((B, 128), jnp.float32),
        pltpu.VMEM((att_len,), jnp.float32),
        pltpu.VMEM_SHARED((NPAD, 128), jnp.float32),
        pltpu.SemaphoreType.DMA,
        pltpu.SemaphoreType.DMA,
        pltpu.SemaphoreType.DMA,
        pltpu.SemaphoreType.DMA,
        pltpu.SemaphoreType.DMA,
        pltpu.SemaphoreType.DMA,
    ]


def _edge1_compute(b, xlb, xrb, val, attb):
    i16 = lax.broadcasted_iota(jnp.int32, (16,), 0)
    p4 = i16 ^ 4
    p2 = i16 ^ 2
    p1 = i16 ^ 1
    for kk in range(4):
        xlv = xlb[b, pl.ds(kk * 16, 16)]
        xrv = xrb[b, pl.ds(kk * 16, 16)]
        sm = xlv + xrv
        e = jnp.maximum(sm, 0.2 * sm)
        p = e * attb[pl.ds(kk * 16, 16)]
        r = p + _lane_perm(p, p4)
        r = r + _lane_perm(r, p2)
        r = r + _lane_perm(r, p1)
        ev = jnp.exp(r)
        val[b, pl.ds(kk * 16, 16)] = ev * xlv
        val[b, pl.ds(64 + kk * 16, 16)] = ev
    return 0


def _sc_edge1_body(*refs):
    _sc_edge_body(_edge1_compute, *refs)


def _sc_edge1(xl, xr, att1f, sd_all):
    mesh = plsc.VectorSubcoreMesh(core_axis_name="c", subcore_axis_name="s")
    f = pl.kernel(
        _sc_edge1_body,
        out_type=jax.ShapeDtypeStruct((2, NPAD, 2 * HC), jnp.float32),
        mesh=mesh,
        scratch_types=_sc_scratch(HC),
    )
    return f(xl, xr, att1f, sd_all)


def _mid_body(acc_ref, b1_ref, wl_ref, bl_ref, wr_ref, br_ref,
              xl2_ref, xr2_ref):
    a0 = acc_ref[0]
    a1 = acc_ref[1]
    num = a0[:, :HC] + a1[:, :HC]
    den = a0[:, HC:] + a1[:, HC:]
    h = num / (den + 1e-16) + b1_ref[...]
    h = jnp.where(h > 0, h, jnp.exp(jnp.minimum(h, 0.0)) - 1.0)
    xl2_ref[...] = jnp.dot(h, wl_ref[...].T,
                           preferred_element_type=jnp.float32) + bl_ref[...]
    xr2_ref[...] = jnp.dot(h, wr_ref[...].T,
                           preferred_element_type=jnp.float32) + br_ref[...]


def _mid(acc1, bias1, wl2p, bl2p, wr2p, br2p):
    blk = 632
    grid = NPAD // blk
    return pl.pallas_call(
        _mid_body,
        grid=(grid,),
        in_specs=[
            pl.BlockSpec((2, blk, 2 * HC), lambda i: (0, i, 0)),
            pl.BlockSpec((1, HC), lambda i: (0, 0)),
            pl.BlockSpec((128, HC), lambda i: (0, 0)),
            pl.BlockSpec((1, 128), lambda i: (0, 0)),
            pl.BlockSpec((128, HC), lambda i: (0, 0)),
            pl.BlockSpec((1, 128), lambda i: (0, 0)),
        ],
        out_specs=[
            pl.BlockSpec((blk, 128), lambda i: (i, 0)),
            pl.BlockSpec((blk, 128), lambda i: (i, 0)),
        ],
        out_shape=[
            jax.ShapeDtypeStruct((NPAD, 128), jnp.float32),
            jax.ShapeDtypeStruct((NPAD, 128), jnp.float32),
        ],
    )(acc1, bias1.reshape(1, HC), wl2p, bl2p.reshape(1, 128),
      wr2p, br2p.reshape(1, 128))


def _edge2_compute(b, xlb, xrb, val, attb):
    i16 = lax.broadcasted_iota(jnp.int32, (16,), 0)
    p8 = i16 ^ 8
    p4 = i16 ^ 4
    p2 = i16 ^ 2
    p1 = i16 ^ 1
    xlv = xlb[b, pl.ds(0, 16)]
    xrv = xrb[b, pl.ds(0, 16)]
    sm = xlv + xrv
    e = jnp.maximum(sm, 0.2 * sm)
    p = e * attb[pl.ds(0, 16)]
    r = p + _lane_perm(p, p8)
    r = r + _lane_perm(r, p4)
    r = r + _lane_perm(r, p2)
    r = r + _lane_perm(r, p1)
    val[b, pl.ds(0, 16)] = jnp.exp(r) * xlv
    return 0


def _sc_edge2_body(*refs):
    _sc_edge_body(_edge2_compute, *refs)


def _sc_edge2(xl2, xr2, att2p, sd_all):
    mesh = plsc.VectorSubcoreMesh(core_axis_name="c", subcore_axis_name="s")
    f = pl.kernel(
        _sc_edge2_body,
        out_type=jax.ShapeDtypeStruct((2, NPAD, 128), jnp.float32),
        mesh=mesh,
        scratch_types=_sc_scratch(16),
    )
    return f(xl2, xr2, att2p, sd_all)


def _final_body(acc_ref, b2_ref, out_ref):
    a = acc_ref[0] + acc_ref[1]
    o = a[:, :7] / (a[:, 7:8] + 1e-16) + b2_ref[...]
    m = jnp.max(o, axis=1, keepdims=True)
    ls = m + jnp.log(jnp.sum(jnp.exp(o - m), axis=1, keepdims=True))
    out_ref[...] = o - ls


def _final(acc2, bias2):
    blk = 632
    grid = NPAD // blk
    return pl.pallas_call(
        _final_body,
        grid=(grid,),
        in_specs=[
            pl.BlockSpec((2, blk, 128), lambda i: (0, i, 0)),
            pl.BlockSpec((1, 7), lambda i: (0, 0)),
        ],
        out_specs=pl.BlockSpec((blk, 7), lambda i: (i, 0)),
        out_shape=jax.ShapeDtypeStruct((NPAD, 7), jnp.float32),
    )(acc2, bias2.reshape(1, 7))


def kernel(x, edge_index, Wl1, bl1, Wr1, br1, att1, bias1,
           Wl2, bl2, Wr2, br2, att2, bias2):
    x_pad = jnp.zeros((NPAD, D_FEAT), jnp.float32).at[:N_NODES].set(x)
    ei = edge_index.astype(jnp.int32)
    loop = jnp.arange(N_NODES, dtype=jnp.int32)
    padi = jnp.full((EP - N_EDGES - N_NODES,), N_NODES, jnp.int32)
    src_all = jnp.concatenate([ei[0], loop, padi])
    dst_all = jnp.concatenate([ei[1], loop, padi])
    sd_all = src_all | (dst_all << 14)

    wl1p = jnp.zeros((128, D_FEAT), jnp.float32).at[:HC].set(Wl1)
    wr1p = jnp.zeros((128, D_FEAT), jnp.float32).at[:HC].set(Wr1)
    bl1p = jnp.zeros((128,), jnp.float32).at[:HC].set(bl1)
    br1p = jnp.zeros((128,), jnp.float32).at[:HC].set(br1)
    wl2p = jnp.zeros((128, HC), jnp.float32).at[:7].set(Wl2)
    bl2p = jnp.zeros((128,), jnp.float32).at[:7].set(bl2).at[7].set(1.0)
    wr2p = jnp.zeros((128, HC), jnp.float32).at[:7].set(Wr2)
    br2p = jnp.zeros((128,), jnp.float32).at[:7].set(br2)
    att2p = jnp.zeros((16,), jnp.float32).at[:7].set(att2[0])

    xl1, xr1 = _proj1(x_pad, wl1p, bl1p, wr1p, br1p)
    acc1 = _sc_edge1(xl1, xr1, att1.reshape(HC), sd_all)
    xl2, xr2 = _mid(acc1, bias1, wl2p, bl2p, wr2p, br2p)
    acc2 = _sc_edge2(xl2, xr2, att2p, sd_all)
    out = _final(acc2, bias2)
    return out[:N_NODES]
